# Initial kernel scaffold; baseline (speedup 1.0000x reference)
import jax
import jax.numpy as jnp
from jax.experimental import pallas as pl


def kernel(x, edge_attr, params, edge_index, batch):
    raise NotImplementedError("write your pallas kernel here")


# speedup vs baseline: 1.0218x; 1.0218x over previous
import functools
import jax
import jax.numpy as jnp
from jax import lax
from jax.experimental import pallas as pl
from jax.experimental.pallas import tpu as pltpu

N = 10000
E = 320000
D = 128
DE = 16
H = 256
G = 64
NUM_LAYERS = 4
NUM_TIMESTEPS = 2

BN = 1000
BE = 2000


def _lrelu(v):
    return jnp.where(v >= 0, v, 0.01 * v)


def _elu(v):
    return jnp.where(v >= 0, v, jnp.exp(jnp.minimum(v, 0.0)) - 1.0)


def _mm_body(x_ref, w_ref, b_ref, o_ref, *, act):
    acc = jnp.dot(x_ref[...], w_ref[...], preferred_element_type=jnp.float32)
    acc = acc + b_ref[...]
    if act == "lrelu":
        acc = _lrelu(acc)
    o_ref[...] = acc


def _mm(x, w_t, b, act="none", bn=BN):
    R, K = x.shape
    M = w_t.shape[1]
    grid = R // bn
    return pl.pallas_call(
        functools.partial(_mm_body, act=act),
        grid=(grid,),
        in_specs=[
            pl.BlockSpec((bn, K), lambda i: (i, 0)),
            pl.BlockSpec((K, M), lambda i: (0, 0)),
            pl.BlockSpec((M,), lambda i: (0,)),
        ],
        out_specs=pl.BlockSpec((bn, M), lambda i: (i, 0)),
        out_shape=jax.ShapeDtypeStruct((R, M), jnp.float32),
    )(x, w_t, b)


def _gru_body(hraw_ref, bias_ref, h_ref, wih_ref, whh_ref, bih_ref, bhh_ref,
              o_ref):
    inp = _elu(hraw_ref[...] + bias_ref[...])
    h = h_ref[...]
    gi = jnp.dot(inp, wih_ref[...], preferred_element_type=jnp.float32) + bih_ref[...]
    gh = jnp.dot(h, whh_ref[...], preferred_element_type=jnp.float32) + bhh_ref[...]
    i_r, i_z, i_n = gi[:, :H], gi[:, H:2 * H], gi[:, 2 * H:]
    h_r, h_z, h_n = gh[:, :H], gh[:, H:2 * H], gh[:, 2 * H:]
    r = jax.nn.sigmoid(i_r + h_r)
    z = jax.nn.sigmoid(i_z + h_z)
    n = jnp.tanh(i_n + r * h_n)
    o_ref[...] = jnp.maximum((1.0 - z) * n + z * h, 0.0)


def _gru(hraw, bias, h, p, bn=BN):
    R = hraw.shape[0]
    grid = R // bn
    return pl.pallas_call(
        _gru_body,
        grid=(grid,),
        in_specs=[
            pl.BlockSpec((bn, H), lambda i: (i, 0)),
            pl.BlockSpec((H,), lambda i: (0,)),
            pl.BlockSpec((bn, H), lambda i: (i, 0)),
            pl.BlockSpec((H, 3 * H), lambda i: (0, 0)),
            pl.BlockSpec((H, 3 * H), lambda i: (0, 0)),
            pl.BlockSpec((3 * H,), lambda i: (0,)),
            pl.BlockSpec((3 * H,), lambda i: (0,)),
        ],
        out_specs=pl.BlockSpec((bn, H), lambda i: (i, 0)),
        out_shape=jax.ShapeDtypeStruct((R, H), jnp.float32),
    )(hraw, bias, h, p["w_ih"].T, p["w_hh"].T, p["b_ih"], p["b_hh"])


def _gat_pre_body(x_ref, w_ref, asrc_ref, adst_ref, xs_ref, s_ref, d_ref):
    xs = jnp.dot(x_ref[...], w_ref[...], preferred_element_type=jnp.float32)
    xs_ref[...] = xs
    s_ref[...] = jnp.sum(xs * asrc_ref[...], axis=1, keepdims=True)
    d_ref[...] = jnp.sum(xs * adst_ref[...], axis=1, keepdims=True)


def _gat_pre(x, p, bn=BN):
    R = x.shape[0]
    grid = R // bn
    return pl.pallas_call(
        _gat_pre_body,
        grid=(grid,),
        in_specs=[
            pl.BlockSpec((bn, H), lambda i: (i, 0)),
            pl.BlockSpec((H, H), lambda i: (0, 0)),
            pl.BlockSpec((H,), lambda i: (0,)),
            pl.BlockSpec((H,), lambda i: (0,)),
        ],
        out_specs=[
            pl.BlockSpec((bn, H), lambda i: (i, 0)),
            pl.BlockSpec((bn, 1), lambda i: (i, 0)),
            pl.BlockSpec((bn, 1), lambda i: (i, 0)),
        ],
        out_shape=[
            jax.ShapeDtypeStruct((R, H), jnp.float32),
            jax.ShapeDtypeStruct((R, 1), jnp.float32),
            jax.ShapeDtypeStruct((R, 1), jnp.float32),
        ],
    )(x, p["lin"].T, p["att_src"], p["att_dst"])


def _segment_softmax(alpha, index, num_segments):
    amax = jax.ops.segment_max(alpha, index, num_segments=num_segments)
    amax = jnp.where(jnp.isfinite(amax), amax, 0.0)
    a = jnp.exp(alpha - amax[index])
    asum = jax.ops.segment_sum(a, index, num_segments=num_segments)
    return a / (asum[index] + 1e-16)


def kernel(x, edge_attr, params, edge_index, batch):
    src, dst = edge_index[0], edge_index[1]
    p = params

    x1 = _mm(x, p["lin1_w"].T, p["lin1_b"], act="lrelu")

    gp = p["gate"]
    w1x = gp["lin1"][:, :H]
    w1e = gp["lin1"][:, H:]
    a = _mm(x1, w1x.T, jnp.zeros((H,), jnp.float32))
    m = _mm(x1, gp["lin2"].T, jnp.zeros((H,), jnp.float32))
    b_e = _mm(edge_attr, w1e.T, jnp.zeros((H,), jnp.float32), bn=BE)
    r_i = x1 @ gp["att_r"]

    xj_t = _lrelu(a[src] + b_e)
    alpha = _lrelu(xj_t @ gp["att_l"] + r_i[dst])
    alpha = _segment_softmax(alpha, dst, N)
    msg = m[src] * alpha[:, None]
    hraw = jax.ops.segment_sum(msg, dst, num_segments=N)

    x_cur = _gru(hraw, gp["bias"], x1, p["gru0"])

    for conv_p, gru_p in zip(p["atom_convs"], p["atom_grus"]):
        xs, s, d = _gat_pre(x_cur, conv_p)
        s = s[:, 0]
        d = d[:, 0]
        alpha = _lrelu(s[src] + d[dst])
        alpha = _segment_softmax(alpha, dst, N)
        hraw = jax.ops.segment_sum(xs[src] * alpha[:, None], dst,
                                   num_segments=N)
        x_cur = _gru(hraw, conv_p["bias"], x_cur, gru_p)

    out = jnp.maximum(jax.ops.segment_sum(x_cur, batch, num_segments=G), 0.0)
    mp = p["mol_conv"]
    for _ in range(NUM_TIMESTEPS):
        xs, s, _ = _gat_pre(x_cur, mp)
        s = s[:, 0]
        xd = out @ mp["lin"].T
        d = xd @ mp["att_dst"]
        alpha = _lrelu(s + d[batch])
        alpha = _segment_softmax(alpha, batch, G)
        hraw = jax.ops.segment_sum(xs * alpha[:, None], batch, num_segments=G)
        out = _gru(hraw, mp["bias"], out, p["mol_gru"], bn=G)

    out = out @ p["lin2_w"].T + p["lin2_b"]
    return out.squeeze(-1)
